# baseline (speedup 1.0000x reference)
import functools
import jax
import jax.numpy as jnp
from jax import lax
from jax.experimental import pallas as pl
from jax.experimental.pallas import tpu as pltpu

N = 10000
GENE = 128
EMB = 32
E = 320000
BLK = 2000


def _prep_body(x_ref, wsp_ref, asp_ref, wex_ref, aex_ref, o_ref):
    csp = jnp.dot(wsp_ref[...], asp_ref[...], preferred_element_type=jnp.float32)
    cex = jnp.dot(wex_ref[...], aex_ref[...], preferred_element_type=jnp.float32)
    c = jnp.concatenate([csp, cex], axis=1)
    o_ref[...] = jnp.dot(x_ref[...], c, preferred_element_type=jnp.float32)


def _prep(x, wsp, asp, wex, aex):
    return pl.pallas_call(
        _prep_body,
        grid=(N // BLK,),
        in_specs=[
            pl.BlockSpec((BLK, GENE), lambda i: (i, 0)),
            pl.BlockSpec((GENE, 512), lambda i: (0, 0)),
            pl.BlockSpec((512, 2), lambda i: (0, 0)),
            pl.BlockSpec((GENE, 512), lambda i: (0, 0)),
            pl.BlockSpec((512, 2), lambda i: (0, 0)),
        ],
        out_specs=pl.BlockSpec((BLK, 4), lambda i: (i, 0)),
        out_shape=jax.ShapeDtypeStruct((N, 4), jnp.float32),
    )(x, wsp, asp, wex, aex)


def _mid_body(agg_ref, den_ref, w1_ref, b1_ref, w2_ref, a2_ref, h2_ref, s2_ref):
    a = agg_ref[...] / (den_ref[...] + 1e-16)
    out1 = jnp.dot(a, w1_ref[...], preferred_element_type=jnp.float32) + b1_ref[...]
    h = jnp.where(out1 > 0, out1, jnp.exp(jnp.minimum(out1, 0.0)) - 1.0)
    h2 = jnp.dot(h, w2_ref[...], preferred_element_type=jnp.float32)
    h2_ref[...] = h2
    s2_ref[...] = jnp.dot(h2, a2_ref[...], preferred_element_type=jnp.float32)


def _mid(agg, den, w1, b1, w2, a2):
    return pl.pallas_call(
        _mid_body,
        grid=(N // BLK,),
        in_specs=[
            pl.BlockSpec((BLK, GENE), lambda i: (i, 0)),
            pl.BlockSpec((BLK, 1), lambda i: (i, 0)),
            pl.BlockSpec((GENE, 512), lambda i: (0, 0)),
            pl.BlockSpec((1, 512), lambda i: (0, 0)),
            pl.BlockSpec((512, EMB), lambda i: (0, 0)),
            pl.BlockSpec((EMB, 2), lambda i: (0, 0)),
        ],
        out_specs=[
            pl.BlockSpec((BLK, EMB), lambda i: (i, 0)),
            pl.BlockSpec((BLK, 2), lambda i: (i, 0)),
        ],
        out_shape=[
            jax.ShapeDtypeStruct((N, EMB), jnp.float32),
            jax.ShapeDtypeStruct((N, 2), jnp.float32),
        ],
    )(agg, den, w1, b1, w2, a2)


def _mlp3(z, w0, b0, w1, b1, w2, b2):
    h = jnp.maximum(jnp.dot(z, w0, preferred_element_type=jnp.float32) + b0, 0.0)
    h = jnp.maximum(jnp.dot(h, w1, preferred_element_type=jnp.float32) + b1, 0.0)
    return jnp.dot(h, w2, preferred_element_type=jnp.float32) + b2


def _final_body(asp_ref, dsp_ref, bsp_ref, aex_ref, dex_ref, bex_ref,
                pw0, pb0, pw1, pb1, pw2, pb2,
                mw0, mb0, mw1, mb1, mw2, mb2,
                tw0, tb0, tw1, tb1, tw2, tb2,
                emb_ref, eex_ref, esp_ref, pi_ref, mu_ref, th_ref):
    esp = asp_ref[...] / (dsp_ref[...] + 1e-16) + bsp_ref[...]
    eex = aex_ref[...] / (dex_ref[...] + 1e-16) + bex_ref[...]
    emb = 0.5 * esp + 0.5 * eex
    esp_ref[...] = esp
    eex_ref[...] = eex
    emb_ref[...] = emb
    zp = _mlp3(emb, pw0[...], pb0[...], pw1[...], pb1[...], pw2[...], pb2[...])
    pi_ref[...] = 1.0 / (1.0 + jnp.exp(-zp))
    zm = _mlp3(emb, mw0[...], mb0[...], mw1[...], mb1[...], mw2[...], mb2[...])
    mu_ref[...] = jnp.maximum(zm, 0.0) + jnp.log1p(jnp.exp(-jnp.abs(zm)))
    zt = _mlp3(emb, tw0[...], tb0[...], tw1[...], tb1[...], tw2[...], tb2[...])
    th_ref[...] = jnp.exp(zt)


def _final(asp, dsp, bsp, aex, dex, bex, pi_p, mu_p, th_p):
    full = lambda r, c: pl.BlockSpec((r, c), lambda i: (0, 0))
    row = lambda c: pl.BlockSpec((BLK, c), lambda i: (i, 0))
    dec_specs = []
    for p in (pi_p, mu_p, th_p):
        for l in p:
            dec_specs.append(full(*l['W'].shape))
            dec_specs.append(full(1, l['b'].shape[0]))
    dec_args = []
    for p in (pi_p, mu_p, th_p):
        for l in p:
            dec_args.append(l['W'])
            dec_args.append(l['b'][None, :])
    return pl.pallas_call(
        _final_body,
        grid=(N // BLK,),
        in_specs=[row(EMB), row(1), full(1, EMB), row(EMB), row(1), full(1, EMB)]
        + dec_specs,
        out_specs=[row(EMB), row(EMB), row(EMB), row(GENE), row(GENE), row(GENE)],
        out_shape=[
            jax.ShapeDtypeStruct((N, EMB), jnp.float32),
            jax.ShapeDtypeStruct((N, EMB), jnp.float32),
            jax.ShapeDtypeStruct((N, EMB), jnp.float32),
            jax.ShapeDtypeStruct((N, GENE), jnp.float32),
            jax.ShapeDtypeStruct((N, GENE), jnp.float32),
            jax.ShapeDtypeStruct((N, GENE), jnp.float32),
        ],
    )(asp, dsp, bsp, aex, dex, bex, *dec_args)


def _edge_phase(s, edge_index, feat):
    src = edge_index[0]
    dst = edge_index[1]
    z = s[src, 0] + s[dst, 1]
    w = jnp.exp(jnp.where(z >= 0, z, 0.2 * z))
    denom = jax.ops.segment_sum(w, dst, num_segments=N)
    agg = jax.ops.segment_sum(feat[src] * w[:, None], dst, num_segments=N)
    return denom, agg


def kernel(x, ge1_params, ge2_params, pi_params, mu_params, theta_params,
           expression_edge_index, spatial_edge_index):
    sp1, sp2 = ge1_params
    ex1, ex2 = ge2_params
    asp = jnp.stack([sp1['a_src'], sp1['a_dst']], axis=1)
    aex = jnp.stack([ex1['a_src'], ex1['a_dst']], axis=1)
    S = _prep(x, sp1['W'], asp, ex1['W'], aex)

    den_sp, agg_sp = _edge_phase(S[:, 0:2], spatial_edge_index, x)
    den_ex, agg_ex = _edge_phase(S[:, 2:4], expression_edge_index, x)

    a2sp = jnp.stack([sp2['a_src'], sp2['a_dst']], axis=1)
    a2ex = jnp.stack([ex2['a_src'], ex2['a_dst']], axis=1)
    h2_sp, s2_sp = _mid(agg_sp, den_sp[:, None], sp1['W'], sp1['b'][None, :],
                        sp2['W'], a2sp)
    h2_ex, s2_ex = _mid(agg_ex, den_ex[:, None], ex1['W'], ex1['b'][None, :],
                        ex2['W'], a2ex)

    den2_sp, agg2_sp = _edge_phase(s2_sp, spatial_edge_index, h2_sp)
    den2_ex, agg2_ex = _edge_phase(s2_ex, expression_edge_index, h2_ex)

    emb, eex, esp, pi, mu, th = _final(
        agg2_sp, den2_sp[:, None], sp2['b'][None, :],
        agg2_ex, den2_ex[:, None], ex2['b'][None, :],
        pi_params, mu_params, theta_params)
    return (emb, eex, esp, pi, mu, th)


# speedup vs baseline: 36.2607x; 36.2607x over previous
import functools
import jax
import jax.numpy as jnp
from jax import lax
from jax.experimental import pallas as pl
from jax.experimental.pallas import tpu as pltpu
from jax.experimental.pallas import tpu_sc as plsc

N = 10000
GENE = 128
EMB = 32
E = 320000
BLK = 2000
NSUB = 16
EP = E // NSUB
NPAD = 10240


def _prep_body(x_ref, wsp_ref, asp_ref, wex_ref, aex_ref, o_ref):
    csp = jnp.dot(wsp_ref[...], asp_ref[...], preferred_element_type=jnp.float32)
    cex = jnp.dot(wex_ref[...], aex_ref[...], preferred_element_type=jnp.float32)
    c = jnp.concatenate([csp, cex], axis=1)
    o_ref[...] = jnp.dot(x_ref[...], c, preferred_element_type=jnp.float32)


def _prep(x, wsp, asp, wex, aex):
    return pl.pallas_call(
        _prep_body,
        grid=(N // BLK,),
        in_specs=[
            pl.BlockSpec((BLK, GENE), lambda i: (i, 0)),
            pl.BlockSpec((GENE, 512), lambda i: (0, 0)),
            pl.BlockSpec((512, 2), lambda i: (0, 0)),
            pl.BlockSpec((GENE, 512), lambda i: (0, 0)),
            pl.BlockSpec((512, 2), lambda i: (0, 0)),
        ],
        out_specs=pl.BlockSpec((BLK, 4), lambda i: (i, 0)),
        out_shape=jax.ShapeDtypeStruct((N, 4), jnp.float32),
    )(x, wsp, asp, wex, aex)


def _mid_body(agg_ref, den_ref, w1_ref, b1_ref, w2_ref, a2_ref, h2_ref, s2_ref):
    a = agg_ref[...] / (den_ref[...] + 1e-16)
    out1 = jnp.dot(a, w1_ref[...], preferred_element_type=jnp.float32) + b1_ref[...]
    h = jnp.where(out1 > 0, out1, jnp.exp(jnp.minimum(out1, 0.0)) - 1.0)
    h2 = jnp.dot(h, w2_ref[...], preferred_element_type=jnp.float32)
    h2_ref[...] = h2
    s2_ref[...] = jnp.dot(h2, a2_ref[...], preferred_element_type=jnp.float32)


def _mid(agg, den, w1, b1, w2, a2):
    return pl.pallas_call(
        _mid_body,
        grid=(N // BLK,),
        in_specs=[
            pl.BlockSpec((BLK, GENE), lambda i: (i, 0)),
            pl.BlockSpec((BLK, 1), lambda i: (i, 0)),
            pl.BlockSpec((GENE, 512), lambda i: (0, 0)),
            pl.BlockSpec((1, 512), lambda i: (0, 0)),
            pl.BlockSpec((512, EMB), lambda i: (0, 0)),
            pl.BlockSpec((EMB, 2), lambda i: (0, 0)),
        ],
        out_specs=[
            pl.BlockSpec((BLK, EMB), lambda i: (i, 0)),
            pl.BlockSpec((BLK, 2), lambda i: (i, 0)),
        ],
        out_shape=[
            jax.ShapeDtypeStruct((N, EMB), jnp.float32),
            jax.ShapeDtypeStruct((N, 2), jnp.float32),
        ],
    )(agg, den, w1, b1, w2, a2)


def _mlp3(z, w0, b0, w1, b1, w2, b2):
    h = jnp.maximum(jnp.dot(z, w0, preferred_element_type=jnp.float32) + b0, 0.0)
    h = jnp.maximum(jnp.dot(h, w1, preferred_element_type=jnp.float32) + b1, 0.0)
    return jnp.dot(h, w2, preferred_element_type=jnp.float32) + b2


def _final_body(asp_ref, dsp_ref, bsp_ref, aex_ref, dex_ref, bex_ref,
                pw0, pb0, pw1, pb1, pw2, pb2,
                mw0, mb0, mw1, mb1, mw2, mb2,
                tw0, tb0, tw1, tb1, tw2, tb2,
                emb_ref, eex_ref, esp_ref, pi_ref, mu_ref, th_ref):
    esp = asp_ref[...] / (dsp_ref[...] + 1e-16) + bsp_ref[...]
    eex = aex_ref[...] / (dex_ref[...] + 1e-16) + bex_ref[...]
    emb = 0.5 * esp + 0.5 * eex
    esp_ref[...] = esp
    eex_ref[...] = eex
    emb_ref[...] = emb
    zp = _mlp3(emb, pw0[...], pb0[...], pw1[...], pb1[...], pw2[...], pb2[...])
    pi_ref[...] = 1.0 / (1.0 + jnp.exp(-zp))
    zm = _mlp3(emb, mw0[...], mb0[...], mw1[...], mb1[...], mw2[...], mb2[...])
    mu_ref[...] = jnp.maximum(zm, 0.0) + jnp.log1p(jnp.exp(-jnp.abs(zm)))
    zt = _mlp3(emb, tw0[...], tb0[...], tw1[...], tb1[...], tw2[...], tb2[...])
    th_ref[...] = jnp.exp(zt)


def _final(asp, dsp, bsp, aex, dex, bex, pi_p, mu_p, th_p):
    full = lambda r, c: pl.BlockSpec((r, c), lambda i: (0, 0))
    row = lambda c: pl.BlockSpec((BLK, c), lambda i: (i, 0))
    dec_specs = []
    for p in (pi_p, mu_p, th_p):
        for l in p:
            dec_specs.append(full(*l['W'].shape))
            dec_specs.append(full(1, l['b'].shape[0]))
    dec_args = []
    for p in (pi_p, mu_p, th_p):
        for l in p:
            dec_args.append(l['W'])
            dec_args.append(l['b'][None, :])
    return pl.pallas_call(
        _final_body,
        grid=(N // BLK,),
        in_specs=[row(EMB), row(1), full(1, EMB), row(EMB), row(1), full(1, EMB)]
        + dec_specs,
        out_specs=[row(EMB), row(EMB), row(EMB), row(GENE), row(GENE), row(GENE)],
        out_shape=[
            jax.ShapeDtypeStruct((N, EMB), jnp.float32),
            jax.ShapeDtypeStruct((N, EMB), jnp.float32),
            jax.ShapeDtypeStruct((N, EMB), jnp.float32),
            jax.ShapeDtypeStruct((N, GENE), jnp.float32),
            jax.ShapeDtypeStruct((N, GENE), jnp.float32),
            jax.ShapeDtypeStruct((N, GENE), jnp.float32),
        ],
    )(asp, dsp, bsp, aex, dex, bex, *dec_args)


def _sc_edge(D, chunk):
    nch = EP // chunk
    ngrp = chunk // 16
    STRIPE = 632
    LAST = N - (NSUB - 1) * STRIPE
    mesh = plsc.VectorSubcoreMesh(core_axis_name="c", subcore_axis_name="s",
                                  num_cores=2, num_subcores=NSUB)

    def body(ssp, sex, srcsp, dstsp, srcex, dstex, fsp, fex,
             den_sp_o, agg_sp_o, den_ex_o, agg_ex_o,
             s_src_v, s_dst_v, srcidx_v, dstidx_v, srcc, dstc, wc, rb, zden,
             den_s, agg_s):
        tid = lax.axis_index("s")
        cid = lax.axis_index("c")

        def run(s_hbm, src_hbm, dst_hbm, f_hbm, den_o, agg_o):
            base = tid * EP
            pltpu.sync_copy(s_hbm.at[0], s_src_v)
            pltpu.sync_copy(s_hbm.at[1], s_dst_v)
            pltpu.sync_copy(src_hbm.at[pl.ds(base, EP)], srcidx_v)
            pltpu.sync_copy(dst_hbm.at[pl.ds(base, EP)], dstidx_v)
            zv = jnp.zeros((16,), jnp.float32)
            for k in range(640 // 16):
                zden[pl.ds(k * 16, 16)] = zv
            for r in range(chunk):
                for c in range(D // 16):
                    rb[r, pl.ds(c * 16, 16)] = zv
            pltpu.sync_copy(zden, den_s.at[pl.ds(tid * 640, 640)])

            def zero_rows(start, ln):
                for jj in range(ln // chunk):
                    pltpu.sync_copy(rb, agg_s.at[pl.ds(start + jj * chunk,
                                                       chunk)])
                t = ln % chunk
                if t:
                    pltpu.sync_copy(rb.at[pl.ds(0, t)],
                                    agg_s.at[pl.ds(start + (ln // chunk) * chunk,
                                                   t)])

            @pl.when(tid < NSUB - 1)
            def _():
                zero_rows(tid * STRIPE, STRIPE)

            @pl.when(tid == NSUB - 1)
            def _():
                zero_rows((NSUB - 1) * STRIPE, LAST)
            plsc.subcore_barrier()

            def chunk_body(j, carry):
                o = j * chunk
                for k in range(ngrp):
                    si = srcidx_v[pl.ds(o + k * 16, 16)]
                    di = dstidx_v[pl.ds(o + k * 16, 16)]
                    srcc[pl.ds(k * 16, 16)] = si
                    dstc[pl.ds(k * 16, 16)] = di
                    sv = plsc.load_gather(s_src_v, [si])
                    dv = plsc.load_gather(s_dst_v, [di])
                    z = sv + dv
                    wc[pl.ds(k * 16, 16)] = jnp.exp(
                        jnp.where(z >= 0.0, z, 0.2 * z))
                pltpu.sync_copy(f_hbm.at[srcc], rb)
                for k in range(ngrp):
                    wv16 = wc[pl.ds(k * 16, 16)]
                    for i in range(16):
                        e_ = k * 16 + i
                        wv = wv16[i]
                        for c in range(D // 16):
                            rb[e_, pl.ds(c * 16, 16)] = (
                                rb[e_, pl.ds(c * 16, 16)] * wv)
                pltpu.sync_copy(wc, den_s.at[dstc], add=True)
                pltpu.sync_copy(rb, agg_s.at[dstc], add=True)
                return carry

            lax.fori_loop(0, nch, chunk_body, 0)
            plsc.subcore_barrier()

            pltpu.sync_copy(den_s.at[pl.ds(tid * 640, 640)],
                            den_o.at[pl.ds(tid * 640, 640)])

            @pl.when(tid < NSUB - 1)
            def _():
                pltpu.sync_copy(agg_s.at[pl.ds(tid * STRIPE, STRIPE)],
                                agg_o.at[pl.ds(tid * STRIPE, STRIPE)])

            @pl.when(tid == NSUB - 1)
            def _():
                pltpu.sync_copy(agg_s.at[pl.ds((NSUB - 1) * STRIPE, LAST)],
                                agg_o.at[pl.ds((NSUB - 1) * STRIPE, LAST)])

        @pl.when(cid == 0)
        def _():
            run(ssp, srcsp, dstsp, fsp, den_sp_o, agg_sp_o)

        @pl.when(cid == 1)
        def _():
            run(sex, srcex, dstex, fex, den_ex_o, agg_ex_o)

    f32 = jnp.float32
    return pl.kernel(
        body,
        out_type=[
            jax.ShapeDtypeStruct((NPAD,), f32),
            jax.ShapeDtypeStruct((N, D), f32),
            jax.ShapeDtypeStruct((NPAD,), f32),
            jax.ShapeDtypeStruct((N, D), f32),
        ],
        mesh=mesh,
        compiler_params=pltpu.CompilerParams(needs_layout_passes=False,
                                             use_tc_tiling_on_sc=False),
        scratch_types=[
            pltpu.VMEM((N,), f32),
            pltpu.VMEM((N,), f32),
            pltpu.VMEM((EP,), jnp.int32),
            pltpu.VMEM((EP,), jnp.int32),
            pltpu.VMEM((chunk,), jnp.int32),
            pltpu.VMEM((chunk,), jnp.int32),
            pltpu.VMEM((chunk,), f32),
            pltpu.VMEM((chunk, D), f32),
            pltpu.VMEM((640,), f32),
            pltpu.VMEM_SHARED((NPAD,), f32),
            pltpu.VMEM_SHARED((N, D), f32),
        ],
    )


_sc_edge_l1h = _sc_edge(64, 80)
_sc_edge_l2 = _sc_edge(EMB, 80)


def kernel(x, ge1_params, ge2_params, pi_params, mu_params, theta_params,
           expression_edge_index, spatial_edge_index):
    sp1, sp2 = ge1_params
    ex1, ex2 = ge2_params
    asp = jnp.stack([sp1['a_src'], sp1['a_dst']], axis=1)
    aex = jnp.stack([ex1['a_src'], ex1['a_dst']], axis=1)
    S = _prep(x, sp1['W'], asp, ex1['W'], aex)

    src_sp = spatial_edge_index[0]
    dst_sp = spatial_edge_index[1]
    src_ex = expression_edge_index[0]
    dst_ex = expression_edge_index[1]

    x0 = x[:, 0:64]
    x1 = x[:, 64:128]
    den_sp, agg_sp0, den_ex, agg_ex0 = _sc_edge_l1h(
        S[:, 0:2].T, S[:, 2:4].T, src_sp, dst_sp, src_ex, dst_ex, x0, x0)
    _, agg_sp1, _, agg_ex1 = _sc_edge_l1h(
        S[:, 0:2].T, S[:, 2:4].T, src_sp, dst_sp, src_ex, dst_ex, x1, x1)
    agg_sp = jnp.concatenate([agg_sp0, agg_sp1], axis=1)
    agg_ex = jnp.concatenate([agg_ex0, agg_ex1], axis=1)
    den_sp = den_sp[:N]
    den_ex = den_ex[:N]

    a2sp = jnp.stack([sp2['a_src'], sp2['a_dst']], axis=1)
    a2ex = jnp.stack([ex2['a_src'], ex2['a_dst']], axis=1)
    h2_sp, s2_sp = _mid(agg_sp, den_sp[:, None], sp1['W'], sp1['b'][None, :],
                        sp2['W'], a2sp)
    h2_ex, s2_ex = _mid(agg_ex, den_ex[:, None], ex1['W'], ex1['b'][None, :],
                        ex2['W'], a2ex)

    den2_sp, agg2_sp, den2_ex, agg2_ex = _sc_edge_l2(
        s2_sp.T, s2_ex.T, src_sp, dst_sp, src_ex, dst_ex, h2_sp, h2_ex)
    den2_sp = den2_sp[:N]
    den2_ex = den2_ex[:N]

    emb, eex, esp, pi, mu, th = _final(
        agg2_sp, den2_sp[:, None], sp2['b'][None, :],
        agg2_ex, den2_ex[:, None], ex2['b'][None, :],
        pi_params, mu_params, theta_params)
    return (emb, eex, esp, pi, mu, th)


# speedup vs baseline: 37.7904x; 1.0422x over previous
import functools
import jax
import jax.numpy as jnp
from jax import lax
from jax.experimental import pallas as pl
from jax.experimental.pallas import tpu as pltpu
from jax.experimental.pallas import tpu_sc as plsc

N = 10000
GENE = 128
EMB = 32
E = 320000
BLK = 2000
NSUB = 16
EP = E // NSUB
NPAD = 10240


def _prep_body(x_ref, wsp_ref, asp_ref, wex_ref, aex_ref, o_ref):
    csp = jnp.dot(wsp_ref[...], asp_ref[...], preferred_element_type=jnp.float32)
    cex = jnp.dot(wex_ref[...], aex_ref[...], preferred_element_type=jnp.float32)
    c = jnp.concatenate([csp, cex], axis=1)
    o_ref[...] = jnp.dot(x_ref[...], c, preferred_element_type=jnp.float32)


def _prep(x, wsp, asp, wex, aex):
    return pl.pallas_call(
        _prep_body,
        grid=(N // BLK,),
        in_specs=[
            pl.BlockSpec((BLK, GENE), lambda i: (i, 0)),
            pl.BlockSpec((GENE, 512), lambda i: (0, 0)),
            pl.BlockSpec((512, 2), lambda i: (0, 0)),
            pl.BlockSpec((GENE, 512), lambda i: (0, 0)),
            pl.BlockSpec((512, 2), lambda i: (0, 0)),
        ],
        out_specs=pl.BlockSpec((BLK, 4), lambda i: (i, 0)),
        out_shape=jax.ShapeDtypeStruct((N, 4), jnp.float32),
    )(x, wsp, asp, wex, aex)


def _mid_body(agg_ref, den_ref, w1_ref, b1_ref, w2_ref, a2_ref, h2_ref, s2_ref):
    a = agg_ref[...] / (den_ref[...] + 1e-16)
    out1 = jnp.dot(a, w1_ref[...], preferred_element_type=jnp.float32) + b1_ref[...]
    h = jnp.where(out1 > 0, out1, jnp.exp(jnp.minimum(out1, 0.0)) - 1.0)
    h2 = jnp.dot(h, w2_ref[...], preferred_element_type=jnp.float32)
    h2_ref[...] = h2
    s2_ref[...] = jnp.dot(h2, a2_ref[...], preferred_element_type=jnp.float32)


def _mid(agg, den, w1, b1, w2, a2):
    return pl.pallas_call(
        _mid_body,
        grid=(N // BLK,),
        in_specs=[
            pl.BlockSpec((BLK, GENE), lambda i: (i, 0)),
            pl.BlockSpec((BLK, 1), lambda i: (i, 0)),
            pl.BlockSpec((GENE, 512), lambda i: (0, 0)),
            pl.BlockSpec((1, 512), lambda i: (0, 0)),
            pl.BlockSpec((512, EMB), lambda i: (0, 0)),
            pl.BlockSpec((EMB, 2), lambda i: (0, 0)),
        ],
        out_specs=[
            pl.BlockSpec((BLK, EMB), lambda i: (i, 0)),
            pl.BlockSpec((BLK, 2), lambda i: (i, 0)),
        ],
        out_shape=[
            jax.ShapeDtypeStruct((N, EMB), jnp.float32),
            jax.ShapeDtypeStruct((N, 2), jnp.float32),
        ],
    )(agg, den, w1, b1, w2, a2)


def _mlp3(z, w0, b0, w1, b1, w2, b2):
    h = jnp.maximum(jnp.dot(z, w0, preferred_element_type=jnp.float32) + b0, 0.0)
    h = jnp.maximum(jnp.dot(h, w1, preferred_element_type=jnp.float32) + b1, 0.0)
    return jnp.dot(h, w2, preferred_element_type=jnp.float32) + b2


def _final_body(asp_ref, dsp_ref, bsp_ref, aex_ref, dex_ref, bex_ref,
                pw0, pb0, pw1, pb1, pw2, pb2,
                mw0, mb0, mw1, mb1, mw2, mb2,
                tw0, tb0, tw1, tb1, tw2, tb2,
                emb_ref, eex_ref, esp_ref, pi_ref, mu_ref, th_ref):
    esp = asp_ref[...] / (dsp_ref[...] + 1e-16) + bsp_ref[...]
    eex = aex_ref[...] / (dex_ref[...] + 1e-16) + bex_ref[...]
    emb = 0.5 * esp + 0.5 * eex
    esp_ref[...] = esp
    eex_ref[...] = eex
    emb_ref[...] = emb
    zp = _mlp3(emb, pw0[...], pb0[...], pw1[...], pb1[...], pw2[...], pb2[...])
    pi_ref[...] = 1.0 / (1.0 + jnp.exp(-zp))
    zm = _mlp3(emb, mw0[...], mb0[...], mw1[...], mb1[...], mw2[...], mb2[...])
    mu_ref[...] = jnp.maximum(zm, 0.0) + jnp.log1p(jnp.exp(-jnp.abs(zm)))
    zt = _mlp3(emb, tw0[...], tb0[...], tw1[...], tb1[...], tw2[...], tb2[...])
    th_ref[...] = jnp.exp(zt)


def _final(asp, dsp, bsp, aex, dex, bex, pi_p, mu_p, th_p):
    full = lambda r, c: pl.BlockSpec((r, c), lambda i: (0, 0))
    row = lambda c: pl.BlockSpec((BLK, c), lambda i: (i, 0))
    dec_specs = []
    for p in (pi_p, mu_p, th_p):
        for l in p:
            dec_specs.append(full(*l['W'].shape))
            dec_specs.append(full(1, l['b'].shape[0]))
    dec_args = []
    for p in (pi_p, mu_p, th_p):
        for l in p:
            dec_args.append(l['W'])
            dec_args.append(l['b'][None, :])
    return pl.pallas_call(
        _final_body,
        grid=(N // BLK,),
        in_specs=[row(EMB), row(1), full(1, EMB), row(EMB), row(1), full(1, EMB)]
        + dec_specs,
        out_specs=[row(EMB), row(EMB), row(EMB), row(GENE), row(GENE), row(GENE)],
        out_shape=[
            jax.ShapeDtypeStruct((N, EMB), jnp.float32),
            jax.ShapeDtypeStruct((N, EMB), jnp.float32),
            jax.ShapeDtypeStruct((N, EMB), jnp.float32),
            jax.ShapeDtypeStruct((N, GENE), jnp.float32),
            jax.ShapeDtypeStruct((N, GENE), jnp.float32),
            jax.ShapeDtypeStruct((N, GENE), jnp.float32),
        ],
    )(asp, dsp, bsp, aex, dex, bex, *dec_args)


def _sc_edge(D, chunk):
    nch = EP // chunk
    ngrp = chunk // 16
    STRIPE = 632
    LAST = N - (NSUB - 1) * STRIPE
    mesh = plsc.VectorSubcoreMesh(core_axis_name="c", subcore_axis_name="s",
                                  num_cores=2, num_subcores=NSUB)

    def body(ssp, sex, epsp, epex, fsp, fex,
             den_sp_o, agg_sp_o, den_ex_o, agg_ex_o,
             s_src_v, s_dst_v, pkc, srcc, dstc, wc, rb, zden,
             den_s, agg_s):
        tid = lax.axis_index("s")
        cid = lax.axis_index("c")

        def run(s_hbm, ep_hbm, f_hbm, den_o, agg_o):
            base = tid * EP
            pltpu.sync_copy(s_hbm.at[0], s_src_v)
            pltpu.sync_copy(s_hbm.at[1], s_dst_v)
            zv = jnp.zeros((16,), jnp.float32)
            for k in range(640 // 16):
                zden[pl.ds(k * 16, 16)] = zv
            for r in range(chunk):
                for c in range(D // 16):
                    rb[r, pl.ds(c * 16, 16)] = zv
            pltpu.sync_copy(zden, den_s.at[pl.ds(tid * 640, 640)])

            def zero_rows(start, ln):
                for jj in range(ln // chunk):
                    pltpu.sync_copy(rb, agg_s.at[pl.ds(start + jj * chunk,
                                                       chunk)])
                t = ln % chunk
                if t:
                    pltpu.sync_copy(rb.at[pl.ds(0, t)],
                                    agg_s.at[pl.ds(start + (ln // chunk) * chunk,
                                                   t)])

            @pl.when(tid < NSUB - 1)
            def _():
                zero_rows(tid * STRIPE, STRIPE)

            @pl.when(tid == NSUB - 1)
            def _():
                zero_rows((NSUB - 1) * STRIPE, LAST)
            plsc.subcore_barrier()

            def chunk_body(j, carry):
                o = base + j * chunk
                pltpu.sync_copy(ep_hbm.at[pl.ds(o, chunk)], pkc)
                for k in range(ngrp):
                    pk = pkc[pl.ds(k * 16, 16)]
                    si = jax.lax.shift_right_logical(pk, 14)
                    di = jax.lax.bitwise_and(pk, 16383)
                    srcc[pl.ds(k * 16, 16)] = si
                    dstc[pl.ds(k * 16, 16)] = di
                    sv = plsc.load_gather(s_src_v, [si])
                    dv = plsc.load_gather(s_dst_v, [di])
                    z = sv + dv
                    wc[pl.ds(k * 16, 16)] = jnp.exp(
                        jnp.where(z >= 0.0, z, 0.2 * z))
                pltpu.sync_copy(f_hbm.at[srcc], rb)
                for k in range(ngrp):
                    wv16 = wc[pl.ds(k * 16, 16)]
                    for i in range(16):
                        e_ = k * 16 + i
                        wv = wv16[i]
                        for c in range(D // 16):
                            rb[e_, pl.ds(c * 16, 16)] = (
                                rb[e_, pl.ds(c * 16, 16)] * wv)
                pltpu.sync_copy(wc, den_s.at[dstc], add=True)
                pltpu.sync_copy(rb, agg_s.at[dstc], add=True)
                return carry

            lax.fori_loop(0, nch, chunk_body, 0)
            plsc.subcore_barrier()

            pltpu.sync_copy(den_s.at[pl.ds(tid * 640, 640)],
                            den_o.at[pl.ds(tid * 640, 640)])

            @pl.when(tid < NSUB - 1)
            def _():
                pltpu.sync_copy(agg_s.at[pl.ds(tid * STRIPE, STRIPE)],
                                agg_o.at[pl.ds(tid * STRIPE, STRIPE)])

            @pl.when(tid == NSUB - 1)
            def _():
                pltpu.sync_copy(agg_s.at[pl.ds((NSUB - 1) * STRIPE, LAST)],
                                agg_o.at[pl.ds((NSUB - 1) * STRIPE, LAST)])

        @pl.when(cid == 0)
        def _():
            run(ssp, epsp, fsp, den_sp_o, agg_sp_o)

        @pl.when(cid == 1)
        def _():
            run(sex, epex, fex, den_ex_o, agg_ex_o)

    f32 = jnp.float32
    return pl.kernel(
        body,
        out_type=[
            jax.ShapeDtypeStruct((NPAD,), f32),
            jax.ShapeDtypeStruct((N, D), f32),
            jax.ShapeDtypeStruct((NPAD,), f32),
            jax.ShapeDtypeStruct((N, D), f32),
        ],
        mesh=mesh,
        compiler_params=pltpu.CompilerParams(needs_layout_passes=False,
                                             use_tc_tiling_on_sc=False),
        scratch_types=[
            pltpu.VMEM((N,), f32),
            pltpu.VMEM((N,), f32),
            pltpu.VMEM((chunk,), jnp.int32),
            pltpu.VMEM((chunk,), jnp.int32),
            pltpu.VMEM((chunk,), jnp.int32),
            pltpu.VMEM((chunk,), f32),
            pltpu.VMEM((chunk, D), f32),
            pltpu.VMEM((640,), f32),
            pltpu.VMEM_SHARED((NPAD,), f32),
            pltpu.VMEM_SHARED((N, D), f32),
        ],
    )


_sc_edge_l1 = _sc_edge(GENE, 80)
_sc_edge_l2 = _sc_edge(EMB, 80)


def kernel(x, ge1_params, ge2_params, pi_params, mu_params, theta_params,
           expression_edge_index, spatial_edge_index):
    sp1, sp2 = ge1_params
    ex1, ex2 = ge2_params
    asp = jnp.stack([sp1['a_src'], sp1['a_dst']], axis=1)
    aex = jnp.stack([ex1['a_src'], ex1['a_dst']], axis=1)
    S = _prep(x, sp1['W'], asp, ex1['W'], aex)

    ep_sp = spatial_edge_index[0] * 16384 + spatial_edge_index[1]
    ep_ex = expression_edge_index[0] * 16384 + expression_edge_index[1]

    den_sp, agg_sp, den_ex, agg_ex = _sc_edge_l1(
        S[:, 0:2].T, S[:, 2:4].T, ep_sp, ep_ex, x, x)
    den_sp = den_sp[:N]
    den_ex = den_ex[:N]

    a2sp = jnp.stack([sp2['a_src'], sp2['a_dst']], axis=1)
    a2ex = jnp.stack([ex2['a_src'], ex2['a_dst']], axis=1)
    h2_sp, s2_sp = _mid(agg_sp, den_sp[:, None], sp1['W'], sp1['b'][None, :],
                        sp2['W'], a2sp)
    h2_ex, s2_ex = _mid(agg_ex, den_ex[:, None], ex1['W'], ex1['b'][None, :],
                        ex2['W'], a2ex)

    den2_sp, agg2_sp, den2_ex, agg2_ex = _sc_edge_l2(
        s2_sp.T, s2_ex.T, ep_sp, ep_ex, h2_sp, h2_ex)
    den2_sp = den2_sp[:N]
    den2_ex = den2_ex[:N]

    emb, eex, esp, pi, mu, th = _final(
        agg2_sp, den2_sp[:, None], sp2['b'][None, :],
        agg2_ex, den2_ex[:, None], ex2['b'][None, :],
        pi_params, mu_params, theta_params)
    return (emb, eex, esp, pi, mu, th)


# speedup vs baseline: 45.8294x; 1.2127x over previous
import functools
import jax
import jax.numpy as jnp
from jax import lax
from jax.experimental import pallas as pl
from jax.experimental.pallas import tpu as pltpu
from jax.experimental.pallas import tpu_sc as plsc

N = 10000
GENE = 128
EMB = 32
E = 320000
BLK = 2000
NSUB = 16
EP = E // NSUB
NPAD = 10240


def _prep_body(x_ref, wsp_ref, asp_ref, wex_ref, aex_ref, o_ref):
    csp = jnp.dot(wsp_ref[...], asp_ref[...], preferred_element_type=jnp.float32)
    cex = jnp.dot(wex_ref[...], aex_ref[...], preferred_element_type=jnp.float32)
    c = jnp.concatenate([csp, cex], axis=1)
    o_ref[...] = jnp.dot(x_ref[...], c, preferred_element_type=jnp.float32)


def _prep(x, wsp, asp, wex, aex):
    return pl.pallas_call(
        _prep_body,
        grid=(N // BLK,),
        in_specs=[
            pl.BlockSpec((BLK, GENE), lambda i: (i, 0)),
            pl.BlockSpec((GENE, 512), lambda i: (0, 0)),
            pl.BlockSpec((512, 2), lambda i: (0, 0)),
            pl.BlockSpec((GENE, 512), lambda i: (0, 0)),
            pl.BlockSpec((512, 2), lambda i: (0, 0)),
        ],
        out_specs=pl.BlockSpec((BLK, 4), lambda i: (i, 0)),
        out_shape=jax.ShapeDtypeStruct((N, 4), jnp.float32),
    )(x, wsp, asp, wex, aex)


def _mid_body(agg_ref, den_ref, w1_ref, b1_ref, w2_ref, a2_ref, h2_ref, s2_ref):
    a = agg_ref[...] / (den_ref[...] + 1e-16)
    out1 = jnp.dot(a, w1_ref[...], preferred_element_type=jnp.float32) + b1_ref[...]
    h = jnp.where(out1 > 0, out1, jnp.exp(jnp.minimum(out1, 0.0)) - 1.0)
    h2 = jnp.dot(h, w2_ref[...], preferred_element_type=jnp.float32)
    h2_ref[...] = h2
    s2_ref[...] = jnp.dot(h2, a2_ref[...], preferred_element_type=jnp.float32)


def _mid(agg, den, w1, b1, w2, a2):
    return pl.pallas_call(
        _mid_body,
        grid=(N // BLK,),
        in_specs=[
            pl.BlockSpec((BLK, GENE), lambda i: (i, 0)),
            pl.BlockSpec((BLK, 1), lambda i: (i, 0)),
            pl.BlockSpec((GENE, 512), lambda i: (0, 0)),
            pl.BlockSpec((1, 512), lambda i: (0, 0)),
            pl.BlockSpec((512, EMB), lambda i: (0, 0)),
            pl.BlockSpec((EMB, 2), lambda i: (0, 0)),
        ],
        out_specs=[
            pl.BlockSpec((BLK, EMB), lambda i: (i, 0)),
            pl.BlockSpec((BLK, 2), lambda i: (i, 0)),
        ],
        out_shape=[
            jax.ShapeDtypeStruct((N, EMB), jnp.float32),
            jax.ShapeDtypeStruct((N, 2), jnp.float32),
        ],
    )(agg, den, w1, b1, w2, a2)


def _mlp3(z, w0, b0, w1, b1, w2, b2):
    h = jnp.maximum(jnp.dot(z, w0, preferred_element_type=jnp.float32) + b0, 0.0)
    h = jnp.maximum(jnp.dot(h, w1, preferred_element_type=jnp.float32) + b1, 0.0)
    return jnp.dot(h, w2, preferred_element_type=jnp.float32) + b2


def _final_body(asp_ref, dsp_ref, bsp_ref, aex_ref, dex_ref, bex_ref,
                pw0, pb0, pw1, pb1, pw2, pb2,
                mw0, mb0, mw1, mb1, mw2, mb2,
                tw0, tb0, tw1, tb1, tw2, tb2,
                emb_ref, eex_ref, esp_ref, pi_ref, mu_ref, th_ref):
    esp = asp_ref[...] / (dsp_ref[...] + 1e-16) + bsp_ref[...]
    eex = aex_ref[...] / (dex_ref[...] + 1e-16) + bex_ref[...]
    emb = 0.5 * esp + 0.5 * eex
    esp_ref[...] = esp
    eex_ref[...] = eex
    emb_ref[...] = emb
    zp = _mlp3(emb, pw0[...], pb0[...], pw1[...], pb1[...], pw2[...], pb2[...])
    pi_ref[...] = 1.0 / (1.0 + jnp.exp(-zp))
    zm = _mlp3(emb, mw0[...], mb0[...], mw1[...], mb1[...], mw2[...], mb2[...])
    mu_ref[...] = jnp.maximum(zm, 0.0) + jnp.log1p(jnp.exp(-jnp.abs(zm)))
    zt = _mlp3(emb, tw0[...], tb0[...], tw1[...], tb1[...], tw2[...], tb2[...])
    th_ref[...] = jnp.exp(zt)


def _final(asp, dsp, bsp, aex, dex, bex, pi_p, mu_p, th_p):
    full = lambda r, c: pl.BlockSpec((r, c), lambda i: (0, 0))
    row = lambda c: pl.BlockSpec((BLK, c), lambda i: (i, 0))
    dec_specs = []
    for p in (pi_p, mu_p, th_p):
        for l in p:
            dec_specs.append(full(*l['W'].shape))
            dec_specs.append(full(1, l['b'].shape[0]))
    dec_args = []
    for p in (pi_p, mu_p, th_p):
        for l in p:
            dec_args.append(l['W'])
            dec_args.append(l['b'][None, :])
    return pl.pallas_call(
        _final_body,
        grid=(N // BLK,),
        in_specs=[row(EMB), row(1), full(1, EMB), row(EMB), row(1), full(1, EMB)]
        + dec_specs,
        out_specs=[row(EMB), row(EMB), row(EMB), row(GENE), row(GENE), row(GENE)],
        out_shape=[
            jax.ShapeDtypeStruct((N, EMB), jnp.float32),
            jax.ShapeDtypeStruct((N, EMB), jnp.float32),
            jax.ShapeDtypeStruct((N, EMB), jnp.float32),
            jax.ShapeDtypeStruct((N, GENE), jnp.float32),
            jax.ShapeDtypeStruct((N, GENE), jnp.float32),
            jax.ShapeDtypeStruct((N, GENE), jnp.float32),
        ],
    )(asp, dsp, bsp, aex, dex, bex, *dec_args)


def _sc_edge(D, chunk):
    nch = EP // chunk
    ngrp = chunk // 16
    STRIPE = 632
    LAST = N - (NSUB - 1) * STRIPE
    mesh = plsc.VectorSubcoreMesh(core_axis_name="c", subcore_axis_name="s",
                                  num_cores=2, num_subcores=NSUB)

    def body(ssp, sex, epsp, epex, fsp, fex,
             den_sp_o, agg_sp_o, den_ex_o, agg_ex_o,
             s_src_v, s_dst_v, pkc, srcc0, srcc1, dstc0, dstc1, wc0, wc1,
             rb0, rb1, zden, den_s, agg_s,
             sem_g0, sem_g1, sem_a0, sem_a1, sem_d0, sem_d1):
        sem_g = (sem_g0, sem_g1)
        sem_a = (sem_a0, sem_a1)
        sem_d = (sem_d0, sem_d1)
        srcc = (srcc0, srcc1)
        dstc = (dstc0, dstc1)
        wc = (wc0, wc1)
        rb = (rb0, rb1)
        tid = lax.axis_index("s")
        cid = lax.axis_index("c")

        def run(s_hbm, ep_hbm, f_hbm, den_o, agg_o):
            base = tid * EP
            pltpu.sync_copy(s_hbm.at[0], s_src_v)
            pltpu.sync_copy(s_hbm.at[1], s_dst_v)
            zv = jnp.zeros((16,), jnp.float32)
            for k in range(640 // 16):
                zden[pl.ds(k * 16, 16)] = zv
            for r in range(chunk):
                for c in range(D // 16):
                    rb[0][r, pl.ds(c * 16, 16)] = zv
            pltpu.sync_copy(zden, den_s.at[pl.ds(tid * 640, 640)])

            def zero_rows(start, ln):
                for jj in range(ln // chunk):
                    pltpu.sync_copy(rb[0], agg_s.at[pl.ds(start + jj * chunk,
                                                          chunk)])
                t = ln % chunk
                if t:
                    pltpu.sync_copy(rb[0].at[pl.ds(0, t)],
                                    agg_s.at[pl.ds(start + (ln // chunk) * chunk,
                                                   t)])

            @pl.when(tid < NSUB - 1)
            def _():
                zero_rows(tid * STRIPE, STRIPE)

            @pl.when(tid == NSUB - 1)
            def _():
                zero_rows((NSUB - 1) * STRIPE, LAST)
            plsc.subcore_barrier()

            def unpack_w(jj, slot):
                o = base + jj * chunk
                pltpu.sync_copy(ep_hbm.at[pl.ds(o, chunk)], pkc)
                for k in range(ngrp):
                    pk = pkc[pl.ds(k * 16, 16)]
                    si = jax.lax.shift_right_logical(pk, 14)
                    di = jax.lax.bitwise_and(pk, 16383)
                    srcc[slot][pl.ds(k * 16, 16)] = si
                    dstc[slot][pl.ds(k * 16, 16)] = di
                    sv = plsc.load_gather(s_src_v, [si])
                    dv = plsc.load_gather(s_dst_v, [di])
                    z = sv + dv
                    wc[slot][pl.ds(k * 16, 16)] = jnp.exp(
                        jnp.where(z >= 0.0, z, 0.2 * z))

            def scale(slot):
                for k in range(ngrp):
                    wv16 = wc[slot][pl.ds(k * 16, 16)]
                    for i in range(16):
                        e_ = k * 16 + i
                        wv = wv16[i]
                        for c in range(D // 16):
                            rb[slot][e_, pl.ds(c * 16, 16)] = (
                                rb[slot][e_, pl.ds(c * 16, 16)] * wv)

            def gather_issue(slot):
                pltpu.async_copy(f_hbm.at[srcc[slot]], rb[slot], sem_g[slot])

            def gather_wait(slot):
                pltpu.make_async_copy(f_hbm.at[srcc[slot]], rb[slot],
                                      sem_g[slot]).wait()

            def scatter_issue(slot):
                pltpu.async_copy(wc[slot], den_s.at[dstc[slot]], sem_d[slot],
                                 add=True)
                pltpu.async_copy(rb[slot], agg_s.at[dstc[slot]], sem_a[slot],
                                 add=True)

            def scatter_wait(slot):
                pltpu.make_async_copy(wc[slot], den_s.at[dstc[slot]],
                                      sem_d[slot]).wait()
                pltpu.make_async_copy(rb[slot], agg_s.at[dstc[slot]],
                                      sem_a[slot]).wait()

            unpack_w(0, 0)
            gather_issue(0)

            def pair_body(t, carry):
                gather_wait(0)

                @pl.when(t > 0)
                def _():
                    scatter_wait(1)
                unpack_w(2 * t + 1, 1)
                gather_issue(1)
                scale(0)
                scatter_issue(0)

                gather_wait(1)
                scatter_wait(0)

                @pl.when(t < nch // 2 - 1)
                def _():
                    unpack_w(2 * t + 2, 0)
                    gather_issue(0)
                scale(1)
                scatter_issue(1)
                return carry

            lax.fori_loop(0, nch // 2, pair_body, 0)
            scatter_wait(1)
            plsc.subcore_barrier()

            pltpu.sync_copy(den_s.at[pl.ds(tid * 640, 640)],
                            den_o.at[pl.ds(tid * 640, 640)])

            @pl.when(tid < NSUB - 1)
            def _():
                pltpu.sync_copy(agg_s.at[pl.ds(tid * STRIPE, STRIPE)],
                                agg_o.at[pl.ds(tid * STRIPE, STRIPE)])

            @pl.when(tid == NSUB - 1)
            def _():
                pltpu.sync_copy(agg_s.at[pl.ds((NSUB - 1) * STRIPE, LAST)],
                                agg_o.at[pl.ds((NSUB - 1) * STRIPE, LAST)])

        @pl.when(cid == 0)
        def _():
            run(ssp, epsp, fsp, den_sp_o, agg_sp_o)

        @pl.when(cid == 1)
        def _():
            run(sex, epex, fex, den_ex_o, agg_ex_o)

    f32 = jnp.float32
    return pl.kernel(
        body,
        out_type=[
            jax.ShapeDtypeStruct((NPAD,), f32),
            jax.ShapeDtypeStruct((N, D), f32),
            jax.ShapeDtypeStruct((NPAD,), f32),
            jax.ShapeDtypeStruct((N, D), f32),
        ],
        mesh=mesh,
        compiler_params=pltpu.CompilerParams(needs_layout_passes=False,
                                             use_tc_tiling_on_sc=False),
        scratch_types=[
            pltpu.VMEM((N,), f32),
            pltpu.VMEM((N,), f32),
            pltpu.VMEM((chunk,), jnp.int32),
            pltpu.VMEM((chunk,), jnp.int32),
            pltpu.VMEM((chunk,), jnp.int32),
            pltpu.VMEM((chunk,), jnp.int32),
            pltpu.VMEM((chunk,), jnp.int32),
            pltpu.VMEM((chunk,), f32),
            pltpu.VMEM((chunk,), f32),
            pltpu.VMEM((chunk, D), f32),
            pltpu.VMEM((chunk, D), f32),
            pltpu.VMEM((640,), f32),
            pltpu.VMEM_SHARED((NPAD,), f32),
            pltpu.VMEM_SHARED((N, D), f32),
            pltpu.SemaphoreType.DMA,
            pltpu.SemaphoreType.DMA,
            pltpu.SemaphoreType.DMA,
            pltpu.SemaphoreType.DMA,
            pltpu.SemaphoreType.DMA,
            pltpu.SemaphoreType.DMA,
        ],
    )


_sc_edge_l1 = _sc_edge(GENE, 80)
_sc_edge_l2 = _sc_edge(EMB, 80)


def kernel(x, ge1_params, ge2_params, pi_params, mu_params, theta_params,
           expression_edge_index, spatial_edge_index):
    sp1, sp2 = ge1_params
    ex1, ex2 = ge2_params
    asp = jnp.stack([sp1['a_src'], sp1['a_dst']], axis=1)
    aex = jnp.stack([ex1['a_src'], ex1['a_dst']], axis=1)
    S = _prep(x, sp1['W'], asp, ex1['W'], aex)

    ep_sp = spatial_edge_index[0] * 16384 + spatial_edge_index[1]
    ep_ex = expression_edge_index[0] * 16384 + expression_edge_index[1]

    den_sp, agg_sp, den_ex, agg_ex = _sc_edge_l1(
        S[:, 0:2].T, S[:, 2:4].T, ep_sp, ep_ex, x, x)
    den_sp = den_sp[:N]
    den_ex = den_ex[:N]

    a2sp = jnp.stack([sp2['a_src'], sp2['a_dst']], axis=1)
    a2ex = jnp.stack([ex2['a_src'], ex2['a_dst']], axis=1)
    h2_sp, s2_sp = _mid(agg_sp, den_sp[:, None], sp1['W'], sp1['b'][None, :],
                        sp2['W'], a2sp)
    h2_ex, s2_ex = _mid(agg_ex, den_ex[:, None], ex1['W'], ex1['b'][None, :],
                        ex2['W'], a2ex)

    den2_sp, agg2_sp, den2_ex, agg2_ex = _sc_edge_l2(
        s2_sp.T, s2_ex.T, ep_sp, ep_ex, h2_sp, h2_ex)
    den2_sp = den2_sp[:N]
    den2_ex = den2_ex[:N]

    emb, eex, esp, pi, mu, th = _final(
        agg2_sp, den2_sp[:, None], sp2['b'][None, :],
        agg2_ex, den2_ex[:, None], ex2['b'][None, :],
        pi_params, mu_params, theta_params)
    return (emb, eex, esp, pi, mu, th)


# speedup vs baseline: 56.6941x; 1.2371x over previous
import functools
import jax
import jax.numpy as jnp
from jax import lax
from jax.experimental import pallas as pl
from jax.experimental.pallas import tpu as pltpu
from jax.experimental.pallas import tpu_sc as plsc

N = 10000
GENE = 128
EMB = 32
E = 320000
BLK = 2000
NSUB = 16
EP = E // NSUB
NPAD = 10240


def _prep_body(x_ref, wsp_ref, asp_ref, wex_ref, aex_ref, o_ref):
    csp = jnp.dot(wsp_ref[...], asp_ref[...], preferred_element_type=jnp.float32)
    cex = jnp.dot(wex_ref[...], aex_ref[...], preferred_element_type=jnp.float32)
    c = jnp.concatenate([csp, cex], axis=1)
    o_ref[...] = jnp.dot(x_ref[...], c, preferred_element_type=jnp.float32)


def _prep(x, wsp, asp, wex, aex):
    return pl.pallas_call(
        _prep_body,
        grid=(N // BLK,),
        in_specs=[
            pl.BlockSpec((BLK, GENE), lambda i: (i, 0)),
            pl.BlockSpec((GENE, 512), lambda i: (0, 0)),
            pl.BlockSpec((512, 2), lambda i: (0, 0)),
            pl.BlockSpec((GENE, 512), lambda i: (0, 0)),
            pl.BlockSpec((512, 2), lambda i: (0, 0)),
        ],
        out_specs=pl.BlockSpec((BLK, 4), lambda i: (i, 0)),
        out_shape=jax.ShapeDtypeStruct((N, 4), jnp.float32),
    )(x, wsp, asp, wex, aex)


def _mid_body(agg_ref, den_ref, w1_ref, b1_ref, w2_ref, a2_ref, h2_ref, s2_ref):
    a = agg_ref[...] / (den_ref[...] + 1e-16)
    out1 = jnp.dot(a, w1_ref[...], preferred_element_type=jnp.float32) + b1_ref[...]
    h = jnp.where(out1 > 0, out1, jnp.exp(jnp.minimum(out1, 0.0)) - 1.0)
    h2 = jnp.dot(h, w2_ref[...], preferred_element_type=jnp.float32)
    h2_ref[...] = h2
    s2_ref[...] = jnp.dot(h2, a2_ref[...], preferred_element_type=jnp.float32)


def _mid(agg, den, w1, b1, w2, a2):
    return pl.pallas_call(
        _mid_body,
        grid=(N // BLK,),
        in_specs=[
            pl.BlockSpec((BLK, GENE), lambda i: (i, 0)),
            pl.BlockSpec((BLK, 1), lambda i: (i, 0)),
            pl.BlockSpec((GENE, 512), lambda i: (0, 0)),
            pl.BlockSpec((1, 512), lambda i: (0, 0)),
            pl.BlockSpec((512, EMB), lambda i: (0, 0)),
            pl.BlockSpec((EMB, 2), lambda i: (0, 0)),
        ],
        out_specs=[
            pl.BlockSpec((BLK, EMB), lambda i: (i, 0)),
            pl.BlockSpec((BLK, 2), lambda i: (i, 0)),
        ],
        out_shape=[
            jax.ShapeDtypeStruct((N, EMB), jnp.float32),
            jax.ShapeDtypeStruct((N, 2), jnp.float32),
        ],
    )(agg, den, w1, b1, w2, a2)


def _mlp3(z, w0, b0, w1, b1, w2, b2):
    h = jnp.maximum(jnp.dot(z, w0, preferred_element_type=jnp.float32) + b0, 0.0)
    h = jnp.maximum(jnp.dot(h, w1, preferred_element_type=jnp.float32) + b1, 0.0)
    return jnp.dot(h, w2, preferred_element_type=jnp.float32) + b2


def _final_body(asp_ref, dsp_ref, bsp_ref, aex_ref, dex_ref, bex_ref,
                pw0, pb0, pw1, pb1, pw2, pb2,
                mw0, mb0, mw1, mb1, mw2, mb2,
                tw0, tb0, tw1, tb1, tw2, tb2,
                emb_ref, eex_ref, esp_ref, pi_ref, mu_ref, th_ref):
    esp = asp_ref[...] / (dsp_ref[...] + 1e-16) + bsp_ref[...]
    eex = aex_ref[...] / (dex_ref[...] + 1e-16) + bex_ref[...]
    emb = 0.5 * esp + 0.5 * eex
    esp_ref[...] = esp
    eex_ref[...] = eex
    emb_ref[...] = emb
    zp = _mlp3(emb, pw0[...], pb0[...], pw1[...], pb1[...], pw2[...], pb2[...])
    pi_ref[...] = 1.0 / (1.0 + jnp.exp(-zp))
    zm = _mlp3(emb, mw0[...], mb0[...], mw1[...], mb1[...], mw2[...], mb2[...])
    mu_ref[...] = jnp.maximum(zm, 0.0) + jnp.log1p(jnp.exp(-jnp.abs(zm)))
    zt = _mlp3(emb, tw0[...], tb0[...], tw1[...], tb1[...], tw2[...], tb2[...])
    th_ref[...] = jnp.exp(zt)


def _final(asp, dsp, bsp, aex, dex, bex, pi_p, mu_p, th_p):
    full = lambda r, c: pl.BlockSpec((r, c), lambda i: (0, 0))
    row = lambda c: pl.BlockSpec((BLK, c), lambda i: (i, 0))
    dec_specs = []
    for p in (pi_p, mu_p, th_p):
        for l in p:
            dec_specs.append(full(*l['W'].shape))
            dec_specs.append(full(1, l['b'].shape[0]))
    dec_args = []
    for p in (pi_p, mu_p, th_p):
        for l in p:
            dec_args.append(l['W'])
            dec_args.append(l['b'][None, :])
    return pl.pallas_call(
        _final_body,
        grid=(N // BLK,),
        in_specs=[row(EMB), row(1), full(1, EMB), row(EMB), row(1), full(1, EMB)]
        + dec_specs,
        out_specs=[row(EMB), row(EMB), row(EMB), row(GENE), row(GENE), row(GENE)],
        out_shape=[
            jax.ShapeDtypeStruct((N, EMB), jnp.float32),
            jax.ShapeDtypeStruct((N, EMB), jnp.float32),
            jax.ShapeDtypeStruct((N, EMB), jnp.float32),
            jax.ShapeDtypeStruct((N, GENE), jnp.float32),
            jax.ShapeDtypeStruct((N, GENE), jnp.float32),
            jax.ShapeDtypeStruct((N, GENE), jnp.float32),
        ],
    )(asp, dsp, bsp, aex, dex, bex, *dec_args)


def _sc_edge(D, chunk):
    nch = EP // chunk
    ngrp = chunk // 16
    STRIPE = 632
    LAST = N - (NSUB - 1) * STRIPE
    mesh = plsc.VectorSubcoreMesh(core_axis_name="c", subcore_axis_name="s",
                                  num_cores=2, num_subcores=NSUB)

    def body(ssp, sex, epsp, epex, fsp, fex,
             den_sp_o, agg_sp_o, den_ex_o, agg_ex_o,
             s_src_v, s_dst_v, pkc0, pkc1, srcc0, srcc1, dstc0, dstc1,
             wc0, wc1, rb0, rb1, zden, den_s, agg_s,
             sem_g0, sem_g1, sem_a0, sem_a1, sem_d0, sem_d1,
             sem_i0, sem_i1):
        pkc = (pkc0, pkc1)
        sem_i = (sem_i0, sem_i1)
        sem_g = (sem_g0, sem_g1)
        sem_a = (sem_a0, sem_a1)
        sem_d = (sem_d0, sem_d1)
        srcc = (srcc0, srcc1)
        dstc = (dstc0, dstc1)
        wc = (wc0, wc1)
        rb = (rb0, rb1)
        tid = lax.axis_index("s")
        cid = lax.axis_index("c")

        def run(s_hbm, ep_hbm, f_hbm, den_o, agg_o):
            base = tid * EP
            pltpu.sync_copy(s_hbm.at[0], s_src_v)
            pltpu.sync_copy(s_hbm.at[1], s_dst_v)
            zv = jnp.zeros((16,), jnp.float32)
            for k in range(640 // 16):
                zden[pl.ds(k * 16, 16)] = zv
            for r in range(chunk):
                for c in range(D // 16):
                    rb[0][r, pl.ds(c * 16, 16)] = zv
            pltpu.sync_copy(zden, den_s.at[pl.ds(tid * 640, 640)])

            def zero_rows(start, ln):
                for jj in range(ln // chunk):
                    pltpu.sync_copy(rb[0], agg_s.at[pl.ds(start + jj * chunk,
                                                          chunk)])
                t = ln % chunk
                if t:
                    pltpu.sync_copy(rb[0].at[pl.ds(0, t)],
                                    agg_s.at[pl.ds(start + (ln // chunk) * chunk,
                                                   t)])

            @pl.when(tid < NSUB - 1)
            def _():
                zero_rows(tid * STRIPE, STRIPE)

            @pl.when(tid == NSUB - 1)
            def _():
                zero_rows((NSUB - 1) * STRIPE, LAST)
            plsc.subcore_barrier()

            def idx_issue(jj, slot):
                o = base + jj * chunk
                pltpu.async_copy(ep_hbm.at[pl.ds(o, chunk)], pkc[slot],
                                 sem_i[slot])

            def idx_wait(jj, slot):
                o = base + jj * chunk
                pltpu.make_async_copy(ep_hbm.at[pl.ds(o, chunk)], pkc[slot],
                                      sem_i[slot]).wait()

            def unpack_w(jj, slot):
                idx_wait(jj, slot)
                for k in range(ngrp):
                    pk = pkc[slot][pl.ds(k * 16, 16)]
                    si = jax.lax.shift_right_logical(pk, 14)
                    di = jax.lax.bitwise_and(pk, 16383)
                    srcc[slot][pl.ds(k * 16, 16)] = si
                    dstc[slot][pl.ds(k * 16, 16)] = di
                    sv = plsc.load_gather(s_src_v, [si])
                    dv = plsc.load_gather(s_dst_v, [di])
                    z = sv + dv
                    wc[slot][pl.ds(k * 16, 16)] = jnp.exp(
                        jnp.where(z >= 0.0, z, 0.2 * z))

            def scale(slot):
                for k in range(ngrp):
                    wv16 = wc[slot][pl.ds(k * 16, 16)]
                    for i in range(16):
                        e_ = k * 16 + i
                        wv = wv16[i]
                        for c in range(D // 16):
                            rb[slot][e_, pl.ds(c * 16, 16)] = (
                                rb[slot][e_, pl.ds(c * 16, 16)] * wv)

            def gather_issue(slot):
                pltpu.async_copy(f_hbm.at[srcc[slot]], rb[slot], sem_g[slot])

            def gather_wait(slot):
                pltpu.make_async_copy(f_hbm.at[srcc[slot]], rb[slot],
                                      sem_g[slot]).wait()

            def scatter_issue(slot):
                pltpu.async_copy(wc[slot], den_s.at[dstc[slot]], sem_d[slot],
                                 add=True)
                pltpu.async_copy(rb[slot], agg_s.at[dstc[slot]], sem_a[slot],
                                 add=True)

            def scatter_wait(slot):
                pltpu.make_async_copy(wc[slot], den_s.at[dstc[slot]],
                                      sem_d[slot]).wait()
                pltpu.make_async_copy(rb[slot], agg_s.at[dstc[slot]],
                                      sem_a[slot]).wait()

            idx_issue(0, 0)
            idx_issue(1, 1)
            unpack_w(0, 0)
            gather_issue(0)

            def pair_body(t, carry):
                gather_wait(0)

                @pl.when(t > 0)
                def _():
                    scatter_wait(1)
                unpack_w(2 * t + 1, 1)
                gather_issue(1)

                @pl.when(2 * t + 2 < nch)
                def _():
                    idx_issue(2 * t + 2, 0)
                scale(0)
                scatter_issue(0)

                gather_wait(1)
                scatter_wait(0)

                @pl.when(t < nch // 2 - 1)
                def _():
                    unpack_w(2 * t + 2, 0)
                    gather_issue(0)
                    idx_issue(2 * t + 3, 1)
                scale(1)
                scatter_issue(1)
                return carry

            lax.fori_loop(0, nch // 2, pair_body, 0)
            scatter_wait(1)
            plsc.subcore_barrier()

            pltpu.sync_copy(den_s.at[pl.ds(tid * 640, 640)],
                            den_o.at[pl.ds(tid * 640, 640)])

            @pl.when(tid < NSUB - 1)
            def _():
                pltpu.sync_copy(agg_s.at[pl.ds(tid * STRIPE, STRIPE)],
                                agg_o.at[pl.ds(tid * STRIPE, STRIPE)])

            @pl.when(tid == NSUB - 1)
            def _():
                pltpu.sync_copy(agg_s.at[pl.ds((NSUB - 1) * STRIPE, LAST)],
                                agg_o.at[pl.ds((NSUB - 1) * STRIPE, LAST)])

        @pl.when(cid == 0)
        def _():
            run(ssp, epsp, fsp, den_sp_o, agg_sp_o)

        @pl.when(cid == 1)
        def _():
            run(sex, epex, fex, den_ex_o, agg_ex_o)

    f32 = jnp.float32
    return pl.kernel(
        body,
        out_type=[
            jax.ShapeDtypeStruct((NPAD,), f32),
            jax.ShapeDtypeStruct((N, D), f32),
            jax.ShapeDtypeStruct((NPAD,), f32),
            jax.ShapeDtypeStruct((N, D), f32),
        ],
        mesh=mesh,
        compiler_params=pltpu.CompilerParams(needs_layout_passes=False,
                                             use_tc_tiling_on_sc=False),
        scratch_types=[
            pltpu.VMEM((N,), f32),
            pltpu.VMEM((N,), f32),
            pltpu.VMEM((chunk,), jnp.int32),
            pltpu.VMEM((chunk,), jnp.int32),
            pltpu.VMEM((chunk,), jnp.int32),
            pltpu.VMEM((chunk,), jnp.int32),
            pltpu.VMEM((chunk,), jnp.int32),
            pltpu.VMEM((chunk,), jnp.int32),
            pltpu.VMEM((chunk,), f32),
            pltpu.VMEM((chunk,), f32),
            pltpu.VMEM((chunk, D), f32),
            pltpu.VMEM((chunk, D), f32),
            pltpu.VMEM((640,), f32),
            pltpu.VMEM_SHARED((NPAD,), f32),
            pltpu.VMEM_SHARED((N, D), f32),
            pltpu.SemaphoreType.DMA,
            pltpu.SemaphoreType.DMA,
            pltpu.SemaphoreType.DMA,
            pltpu.SemaphoreType.DMA,
            pltpu.SemaphoreType.DMA,
            pltpu.SemaphoreType.DMA,
            pltpu.SemaphoreType.DMA,
            pltpu.SemaphoreType.DMA,
        ],
    )


_sc_edge_l1 = _sc_edge(GENE, 80)
_sc_edge_l2 = _sc_edge(EMB, 80)


def kernel(x, ge1_params, ge2_params, pi_params, mu_params, theta_params,
           expression_edge_index, spatial_edge_index):
    sp1, sp2 = ge1_params
    ex1, ex2 = ge2_params
    asp = jnp.stack([sp1['a_src'], sp1['a_dst']], axis=1)
    aex = jnp.stack([ex1['a_src'], ex1['a_dst']], axis=1)
    S = _prep(x, sp1['W'], asp, ex1['W'], aex)

    ep_sp = spatial_edge_index[0] * 16384 + spatial_edge_index[1]
    ep_ex = expression_edge_index[0] * 16384 + expression_edge_index[1]

    den_sp, agg_sp, den_ex, agg_ex = _sc_edge_l1(
        S[:, 0:2].T, S[:, 2:4].T, ep_sp, ep_ex, x, x)
    den_sp = den_sp[:N]
    den_ex = den_ex[:N]

    a2sp = jnp.stack([sp2['a_src'], sp2['a_dst']], axis=1)
    a2ex = jnp.stack([ex2['a_src'], ex2['a_dst']], axis=1)
    h2_sp, s2_sp = _mid(agg_sp, den_sp[:, None], sp1['W'], sp1['b'][None, :],
                        sp2['W'], a2sp)
    h2_ex, s2_ex = _mid(agg_ex, den_ex[:, None], ex1['W'], ex1['b'][None, :],
                        ex2['W'], a2ex)

    den2_sp, agg2_sp, den2_ex, agg2_ex = _sc_edge_l2(
        s2_sp.T, s2_ex.T, ep_sp, ep_ex, h2_sp, h2_ex)
    den2_sp = den2_sp[:N]
    den2_ex = den2_ex[:N]

    emb, eex, esp, pi, mu, th = _final(
        agg2_sp, den2_sp[:, None], sp2['b'][None, :],
        agg2_ex, den2_ex[:, None], ex2['b'][None, :],
        pi_params, mu_params, theta_params)
    return (emb, eex, esp, pi, mu, th)


# speedup vs baseline: 67.1469x; 1.1844x over previous
import functools
import jax
import jax.numpy as jnp
from jax import lax
from jax.experimental import pallas as pl
from jax.experimental.pallas import tpu as pltpu
from jax.experimental.pallas import tpu_sc as plsc

N = 10000
GENE = 128
EMB = 32
E = 320000
BLK = 2000
NSUB = 16
EP = E // NSUB
NPAD = 10240
NJUNK = 10048
EPT2 = 10240
EPAD = 32 * EPT2 - E


def _prep_body(x_ref, wsp_ref, asp_ref, wex_ref, aex_ref, o_ref):
    csp = jnp.dot(wsp_ref[...], asp_ref[...], preferred_element_type=jnp.float32)
    cex = jnp.dot(wex_ref[...], aex_ref[...], preferred_element_type=jnp.float32)
    c = jnp.concatenate([csp, cex], axis=1)
    o_ref[...] = jnp.dot(x_ref[...], c, preferred_element_type=jnp.float32)


def _prep(x, wsp, asp, wex, aex):
    return pl.pallas_call(
        _prep_body,
        grid=(N // BLK,),
        in_specs=[
            pl.BlockSpec((BLK, GENE), lambda i: (i, 0)),
            pl.BlockSpec((GENE, 512), lambda i: (0, 0)),
            pl.BlockSpec((512, 2), lambda i: (0, 0)),
            pl.BlockSpec((GENE, 512), lambda i: (0, 0)),
            pl.BlockSpec((512, 2), lambda i: (0, 0)),
        ],
        out_specs=pl.BlockSpec((BLK, 4), lambda i: (i, 0)),
        out_shape=jax.ShapeDtypeStruct((N, 4), jnp.float32),
    )(x, wsp, asp, wex, aex)


def _mid_body(aggs_ref, dens_ref, w1s_ref, b1s_ref, w2s_ref, a2s_ref,
              agge_ref, dene_ref, w1e_ref, b1e_ref, w2e_ref, a2e_ref,
              h2s_ref, s2s_ref, h2e_ref, s2e_ref):
    def enc(agg_ref, den_ref, w1_ref, b1_ref, w2_ref, a2_ref, h2_ref, s2_ref):
        a = agg_ref[...] / (den_ref[...] + 1e-16)
        out1 = jnp.dot(a, w1_ref[...],
                       preferred_element_type=jnp.float32) + b1_ref[...]
        h = jnp.where(out1 > 0, out1, jnp.exp(jnp.minimum(out1, 0.0)) - 1.0)
        h2 = jnp.dot(h, w2_ref[...], preferred_element_type=jnp.float32)
        h2_ref[...] = h2
        s2_ref[...] = jnp.dot(h2, a2_ref[...],
                              preferred_element_type=jnp.float32)
    enc(aggs_ref, dens_ref, w1s_ref, b1s_ref, w2s_ref, a2s_ref,
        h2s_ref, s2s_ref)
    enc(agge_ref, dene_ref, w1e_ref, b1e_ref, w2e_ref, a2e_ref,
        h2e_ref, s2e_ref)


def _mid(aggs, dens, w1s, b1s, w2s, a2s, agge, dene, w1e, b1e, w2e, a2e):
    enc_in = [
        pl.BlockSpec((BLK, GENE), lambda i: (i, 0)),
        pl.BlockSpec((BLK, 1), lambda i: (i, 0)),
        pl.BlockSpec((GENE, 512), lambda i: (0, 0)),
        pl.BlockSpec((1, 512), lambda i: (0, 0)),
        pl.BlockSpec((512, EMB), lambda i: (0, 0)),
        pl.BlockSpec((EMB, 2), lambda i: (0, 0)),
    ]
    enc_out = [
        pl.BlockSpec((BLK, EMB), lambda i: (i, 0)),
        pl.BlockSpec((BLK, 2), lambda i: (i, 0)),
    ]
    return pl.pallas_call(
        _mid_body,
        grid=(N // BLK,),
        in_specs=enc_in + enc_in,
        out_specs=enc_out + enc_out,
        out_shape=[
            jax.ShapeDtypeStruct((N, EMB), jnp.float32),
            jax.ShapeDtypeStruct((N, 2), jnp.float32),
            jax.ShapeDtypeStruct((N, EMB), jnp.float32),
            jax.ShapeDtypeStruct((N, 2), jnp.float32),
        ],
    )(aggs, dens, w1s, b1s, w2s, a2s, agge, dene, w1e, b1e, w2e, a2e)


def _mlp3(z, w0, b0, w1, b1, w2, b2):
    h = jnp.maximum(jnp.dot(z, w0, preferred_element_type=jnp.float32) + b0, 0.0)
    h = jnp.maximum(jnp.dot(h, w1, preferred_element_type=jnp.float32) + b1, 0.0)
    return jnp.dot(h, w2, preferred_element_type=jnp.float32) + b2


def _final_body(asp_ref, dsp_ref, bsp_ref, aex_ref, dex_ref, bex_ref,
                pw0, pb0, pw1, pb1, pw2, pb2,
                mw0, mb0, mw1, mb1, mw2, mb2,
                tw0, tb0, tw1, tb1, tw2, tb2,
                emb_ref, eex_ref, esp_ref, pi_ref, mu_ref, th_ref):
    esp = asp_ref[...] / (dsp_ref[...] + 1e-16) + bsp_ref[...]
    eex = aex_ref[...] / (dex_ref[...] + 1e-16) + bex_ref[...]
    emb = 0.5 * esp + 0.5 * eex
    esp_ref[...] = esp
    eex_ref[...] = eex
    emb_ref[...] = emb
    zp = _mlp3(emb, pw0[...], pb0[...], pw1[...], pb1[...], pw2[...], pb2[...])
    pi_ref[...] = 1.0 / (1.0 + jnp.exp(-zp))
    zm = _mlp3(emb, mw0[...], mb0[...], mw1[...], mb1[...], mw2[...], mb2[...])
    mu_ref[...] = jnp.maximum(zm, 0.0) + jnp.log1p(jnp.exp(-jnp.abs(zm)))
    zt = _mlp3(emb, tw0[...], tb0[...], tw1[...], tb1[...], tw2[...], tb2[...])
    th_ref[...] = jnp.exp(zt)


def _final(asp, dsp, bsp, aex, dex, bex, pi_p, mu_p, th_p):
    full = lambda r, c: pl.BlockSpec((r, c), lambda i: (0, 0))
    row = lambda c: pl.BlockSpec((BLK, c), lambda i: (i, 0))
    dec_specs = []
    for p in (pi_p, mu_p, th_p):
        for l in p:
            dec_specs.append(full(*l['W'].shape))
            dec_specs.append(full(1, l['b'].shape[0]))
    dec_args = []
    for p in (pi_p, mu_p, th_p):
        for l in p:
            dec_args.append(l['W'])
            dec_args.append(l['b'][None, :])
    return pl.pallas_call(
        _final_body,
        grid=(N // BLK,),
        in_specs=[row(EMB), row(1), full(1, EMB), row(EMB), row(1), full(1, EMB)]
        + dec_specs,
        out_specs=[row(EMB), row(EMB), row(EMB), row(GENE), row(GENE), row(GENE)],
        out_shape=[
            jax.ShapeDtypeStruct((N, EMB), jnp.float32),
            jax.ShapeDtypeStruct((N, EMB), jnp.float32),
            jax.ShapeDtypeStruct((N, EMB), jnp.float32),
            jax.ShapeDtypeStruct((N, GENE), jnp.float32),
            jax.ShapeDtypeStruct((N, GENE), jnp.float32),
            jax.ShapeDtypeStruct((N, GENE), jnp.float32),
        ],
    )(asp, dsp, bsp, aex, dex, bex, *dec_args)


def _sc_edge(D, chunk, ept):
    nch = ept // chunk
    ngrp = chunk // 16
    STRIPE = 632
    LAST = N - (NSUB - 1) * STRIPE
    mesh = plsc.VectorSubcoreMesh(core_axis_name="c", subcore_axis_name="s",
                                  num_cores=2, num_subcores=NSUB)

    def body(ssp, sex, epsp, epex, fsp, fex,
             den_sp_o, agg_sp_o, den_ex_o, agg_ex_o,
             s_src_v, s_dst_v, pkc0, pkc1, srcc0, srcc1, dstc0, dstc1,
             wc0, wc1, rb0, rb1, zden, den_s, agg_s,
             sem_g0, sem_g1, sem_a0, sem_a1, sem_d0, sem_d1,
             sem_i0, sem_i1):
        pkc = (pkc0, pkc1)
        sem_i = (sem_i0, sem_i1)
        sem_g = (sem_g0, sem_g1)
        sem_a = (sem_a0, sem_a1)
        sem_d = (sem_d0, sem_d1)
        srcc = (srcc0, srcc1)
        dstc = (dstc0, dstc1)
        wc = (wc0, wc1)
        rb = (rb0, rb1)
        tid = lax.axis_index("s")
        cid = lax.axis_index("c")

        def run(s_hbm, ep_hbm, f_hbm, den_o, agg_o):
            base = tid * ept
            pltpu.sync_copy(s_hbm.at[0], s_src_v)
            pltpu.sync_copy(s_hbm.at[1], s_dst_v.at[pl.ds(0, N)])
            zv = jnp.zeros((16,), jnp.float32)
            for k in range(640 // 16):
                zden[pl.ds(k * 16, 16)] = zv
            for r in range(chunk):
                for c in range(D // 16):
                    rb[0][r, pl.ds(c * 16, 16)] = zv
            pltpu.sync_copy(zden, den_s.at[pl.ds(tid * 640, 640)])

            def zero_rows(start, ln):
                for jj in range(ln // chunk):
                    pltpu.sync_copy(rb[0], agg_s.at[pl.ds(start + jj * chunk,
                                                          chunk)])
                t = ln % chunk
                if t:
                    pltpu.sync_copy(rb[0].at[pl.ds(0, t)],
                                    agg_s.at[pl.ds(start + (ln // chunk) * chunk,
                                                   t)])

            @pl.when(tid < NSUB - 1)
            def _():
                zero_rows(tid * STRIPE, STRIPE)

            @pl.when(tid == NSUB - 1)
            def _():
                zero_rows((NSUB - 1) * STRIPE, LAST)
            plsc.subcore_barrier()

            def idx_issue(jj, slot):
                o = base + jj * chunk
                pltpu.async_copy(ep_hbm.at[pl.ds(o, chunk)], pkc[slot],
                                 sem_i[slot])

            def idx_wait(jj, slot):
                o = base + jj * chunk
                pltpu.make_async_copy(ep_hbm.at[pl.ds(o, chunk)], pkc[slot],
                                      sem_i[slot]).wait()

            def unpack_w(jj, slot):
                idx_wait(jj, slot)
                for k in range(ngrp):
                    pk = pkc[slot][pl.ds(k * 16, 16)]
                    si = jax.lax.shift_right_logical(pk, 14)
                    di = jax.lax.bitwise_and(pk, 16383)
                    srcc[slot][pl.ds(k * 16, 16)] = si
                    dstc[slot][pl.ds(k * 16, 16)] = di
                    sv = plsc.load_gather(s_src_v, [si])
                    dv = plsc.load_gather(s_dst_v, [di])
                    z = sv + dv
                    wc[slot][pl.ds(k * 16, 16)] = jnp.exp(
                        jnp.where(z >= 0.0, z, 0.2 * z))

            def scale(slot):
                for k in range(ngrp):
                    wv16 = wc[slot][pl.ds(k * 16, 16)]
                    for i in range(16):
                        e_ = k * 16 + i
                        wv = wv16[i]
                        for c in range(D // 16):
                            rb[slot][e_, pl.ds(c * 16, 16)] = (
                                rb[slot][e_, pl.ds(c * 16, 16)] * wv)

            def gather_issue(slot):
                pltpu.async_copy(f_hbm.at[srcc[slot]], rb[slot], sem_g[slot])

            def gather_wait(slot):
                pltpu.make_async_copy(f_hbm.at[srcc[slot]], rb[slot],
                                      sem_g[slot]).wait()

            def scatter_issue(slot):
                pltpu.async_copy(wc[slot], den_s.at[dstc[slot]], sem_d[slot],
                                 add=True)
                pltpu.async_copy(rb[slot], agg_s.at[dstc[slot]], sem_a[slot],
                                 add=True)

            def scatter_wait(slot):
                pltpu.make_async_copy(wc[slot], den_s.at[dstc[slot]],
                                      sem_d[slot]).wait()
                pltpu.make_async_copy(rb[slot], agg_s.at[dstc[slot]],
                                      sem_a[slot]).wait()

            idx_issue(0, 0)
            idx_issue(1, 1)
            unpack_w(0, 0)
            gather_issue(0)

            def pair_body(t, carry):
                gather_wait(0)

                @pl.when(t > 0)
                def _():
                    scatter_wait(1)
                unpack_w(2 * t + 1, 1)
                gather_issue(1)

                @pl.when(2 * t + 2 < nch)
                def _():
                    idx_issue(2 * t + 2, 0)
                scale(0)
                scatter_issue(0)

                gather_wait(1)
                scatter_wait(0)

                @pl.when(t < nch // 2 - 1)
                def _():
                    unpack_w(2 * t + 2, 0)
                    gather_issue(0)
                    idx_issue(2 * t + 3, 1)
                scale(1)
                scatter_issue(1)
                return carry

            lax.fori_loop(0, nch // 2, pair_body, 0)
            scatter_wait(1)
            plsc.subcore_barrier()

            pltpu.sync_copy(den_s.at[pl.ds(tid * 640, 640)],
                            den_o.at[pl.ds(tid * 640, 640)])

            @pl.when(tid < NSUB - 1)
            def _():
                pltpu.sync_copy(agg_s.at[pl.ds(tid * STRIPE, STRIPE)],
                                agg_o.at[pl.ds(tid * STRIPE, STRIPE)])

            @pl.when(tid == NSUB - 1)
            def _():
                pltpu.sync_copy(agg_s.at[pl.ds((NSUB - 1) * STRIPE, LAST)],
                                agg_o.at[pl.ds((NSUB - 1) * STRIPE, LAST)])

        @pl.when(cid == 0)
        def _():
            run(ssp, epsp, fsp, den_sp_o, agg_sp_o)

        @pl.when(cid == 1)
        def _():
            run(sex, epex, fex, den_ex_o, agg_ex_o)

    f32 = jnp.float32
    return pl.kernel(
        body,
        out_type=[
            jax.ShapeDtypeStruct((NPAD,), f32),
            jax.ShapeDtypeStruct((N, D), f32),
            jax.ShapeDtypeStruct((NPAD,), f32),
            jax.ShapeDtypeStruct((N, D), f32),
        ],
        mesh=mesh,
        compiler_params=pltpu.CompilerParams(needs_layout_passes=False,
                                             use_tc_tiling_on_sc=False),
        scratch_types=[
            pltpu.VMEM((N,), f32),
            pltpu.VMEM((NJUNK,), f32),
            pltpu.VMEM((chunk,), jnp.int32),
            pltpu.VMEM((chunk,), jnp.int32),
            pltpu.VMEM((chunk,), jnp.int32),
            pltpu.VMEM((chunk,), jnp.int32),
            pltpu.VMEM((chunk,), jnp.int32),
            pltpu.VMEM((chunk,), jnp.int32),
            pltpu.VMEM((chunk,), f32),
            pltpu.VMEM((chunk,), f32),
            pltpu.VMEM((chunk, D), f32),
            pltpu.VMEM((chunk, D), f32),
            pltpu.VMEM((640,), f32),
            pltpu.VMEM_SHARED((NPAD,), f32),
            pltpu.VMEM_SHARED((NJUNK, D), f32),
            pltpu.SemaphoreType.DMA,
            pltpu.SemaphoreType.DMA,
            pltpu.SemaphoreType.DMA,
            pltpu.SemaphoreType.DMA,
            pltpu.SemaphoreType.DMA,
            pltpu.SemaphoreType.DMA,
            pltpu.SemaphoreType.DMA,
            pltpu.SemaphoreType.DMA,
        ],
    )


_sc_edge_l1 = _sc_edge(GENE, 80, EP)
_sc_edge_l2 = _sc_edge(EMB, 128, EPT2)


def kernel(x, ge1_params, ge2_params, pi_params, mu_params, theta_params,
           expression_edge_index, spatial_edge_index):
    sp1, sp2 = ge1_params
    ex1, ex2 = ge2_params
    asp = jnp.stack([sp1['a_src'], sp1['a_dst']], axis=1)
    aex = jnp.stack([ex1['a_src'], ex1['a_dst']], axis=1)
    S = _prep(x, sp1['W'], asp, ex1['W'], aex)

    pad = jnp.full((EPAD,), N, dtype=jnp.int32)
    ep_sp = jnp.concatenate(
        [spatial_edge_index[0] * 16384 + spatial_edge_index[1], pad])
    ep_ex = jnp.concatenate(
        [expression_edge_index[0] * 16384 + expression_edge_index[1], pad])

    den_sp, agg_sp, den_ex, agg_ex = _sc_edge_l1(
        S[:, 0:2].T, S[:, 2:4].T, ep_sp, ep_ex, x, x)
    den_sp = den_sp[:N]
    den_ex = den_ex[:N]

    a2sp = jnp.stack([sp2['a_src'], sp2['a_dst']], axis=1)
    a2ex = jnp.stack([ex2['a_src'], ex2['a_dst']], axis=1)
    h2_sp, s2_sp, h2_ex, s2_ex = _mid(
        agg_sp, den_sp[:, None], sp1['W'], sp1['b'][None, :], sp2['W'], a2sp,
        agg_ex, den_ex[:, None], ex1['W'], ex1['b'][None, :], ex2['W'], a2ex)

    den2_sp, agg2_sp, den2_ex, agg2_ex = _sc_edge_l2(
        s2_sp.T, s2_ex.T, ep_sp, ep_ex, h2_sp, h2_ex)
    den2_sp = den2_sp[:N]
    den2_ex = den2_ex[:N]

    emb, eex, esp, pi, mu, th = _final(
        agg2_sp, den2_sp[:, None], sp2['b'][None, :],
        agg2_ex, den2_ex[:, None], ex2['b'][None, :],
        pi_params, mu_params, theta_params)
    return (emb, eex, esp, pi, mu, th)
